# baseline (speedup 1.0000x reference)
import functools

import jax
import jax.numpy as jnp
from jax.experimental import pallas as pl
from jax.experimental.pallas import tpu as pltpu

T = 128
H = 1024
FF = 2048
E = 16
HH = H // 2
FH = FF // 2


def _moe_body(rl_ref, x_ref, a0_ref, a1_ref, b0_ref, b1_ref, c0_ref, c1_ref,
              out_ref, w_ref):
    e = pl.program_id(0)

    @pl.when(e == 0)
    def _prologue():
        logits = rl_ref[...]
        lane = jax.lax.broadcasted_iota(jnp.int32, (T, E), 1)
        neg = jnp.float32(jnp.finfo(jnp.float32).min)
        m1 = jnp.max(logits, axis=-1, keepdims=True)
        i1 = jnp.min(jnp.where(logits == m1, lane, E), axis=-1, keepdims=True)
        masked = jnp.where(lane == i1, neg, logits)
        m2 = jnp.max(masked, axis=-1, keepdims=True)
        i2 = jnp.min(jnp.where(masked == m2, lane, E), axis=-1, keepdims=True)
        w1 = 1.0 / (1.0 + jnp.exp(m2 - m1))
        w2 = 1.0 - w1
        w_ref[...] = (jnp.where(lane == i1, w1, 0.0)
                      + jnp.where(lane == i2, w2, 0.0))
        out_ref[...] = jnp.zeros_like(out_ref)

    x0 = x_ref[:, :HH]
    x1 = x_ref[:, HH:]
    g = (jnp.dot(x0, a0_ref[0, 0], preferred_element_type=jnp.float32)
         + jnp.dot(x1, a1_ref[0, 0], preferred_element_type=jnp.float32))
    u = (jnp.dot(x0, b0_ref[0, 0], preferred_element_type=jnp.float32)
         + jnp.dot(x1, b1_ref[0, 0], preferred_element_type=jnp.float32))
    h = jax.nn.silu(g) * u
    ye = (jnp.dot(h[:, :FH], c0_ref[0, 0], preferred_element_type=jnp.float32)
          + jnp.dot(h[:, FH:], c1_ref[0, 0], preferred_element_type=jnp.float32))

    lane = jax.lax.broadcasted_iota(jnp.int32, (T, E), 1)
    w_e = jnp.sum(jnp.where(lane == e, w_ref[...], 0.0), axis=-1, keepdims=True)
    out_ref[...] += ye * w_e


@functools.partial(jax.jit)
def kernel(x, router_logits, wi_0, wi_1, wo):
    wi_0h = wi_0.reshape(E, 2, HH, FF)
    wi_1h = wi_1.reshape(E, 2, HH, FF)
    woh = wo.reshape(E, 2, FH, H)
    wi_spec0 = pl.BlockSpec((1, 1, HH, FF), lambda e: (e, 0, 0, 0))
    wi_spec1 = pl.BlockSpec((1, 1, HH, FF), lambda e: (e, 1, 0, 0))
    wo_spec0 = pl.BlockSpec((1, 1, FH, H), lambda e: (e, 0, 0, 0))
    wo_spec1 = pl.BlockSpec((1, 1, FH, H), lambda e: (e, 1, 0, 0))
    return pl.pallas_call(
        _moe_body,
        grid=(E,),
        in_specs=[
            pl.BlockSpec((T, E), lambda e: (0, 0)),
            pl.BlockSpec((T, H), lambda e: (0, 0)),
            wi_spec0, wi_spec1, wi_spec0, wi_spec1, wo_spec0, wo_spec1,
        ],
        out_specs=pl.BlockSpec((T, H), lambda e: (0, 0)),
        out_shape=jax.ShapeDtypeStruct((T, H), jnp.float32),
        scratch_shapes=[pltpu.VMEM((T, E), jnp.float32)],
    )(router_logits, x, wi_0h, wi_0h, wi_1h, wi_1h, woh, woh)


# speedup vs baseline: 1.0228x; 1.0228x over previous
import functools

import jax
import jax.numpy as jnp
from jax.experimental import pallas as pl
from jax.experimental.pallas import tpu as pltpu

T = 128
H = 1024
FF = 2048
E = 16
NC = 2
EC = E // NC
BF = 1024
NF = FF // BF


def _moe_body(rl_ref, x_ref, wi0_ref, wi1_ref, wo_ref, out_ref, w_ref):
    c = pl.program_id(0)
    e8 = pl.program_id(1)
    f = pl.program_id(2)
    e = c * EC + e8

    @pl.when((e8 == 0) & (f == 0))
    def _prologue():
        logits = rl_ref[...]
        lane = jax.lax.broadcasted_iota(jnp.int32, (T, E), 1)
        neg = jnp.float32(jnp.finfo(jnp.float32).min)
        m1 = jnp.max(logits, axis=-1, keepdims=True)
        i1 = jnp.min(jnp.where(logits == m1, lane, E), axis=-1, keepdims=True)
        masked = jnp.where(lane == i1, neg, logits)
        m2 = jnp.max(masked, axis=-1, keepdims=True)
        i2 = jnp.min(jnp.where(masked == m2, lane, E), axis=-1, keepdims=True)
        w1 = 1.0 / (1.0 + jnp.exp(m2 - m1))
        w2 = 1.0 - w1
        w_ref[...] = (jnp.where(lane == i1, w1, 0.0)
                      + jnp.where(lane == i2, w2, 0.0))
        out_ref[...] = jnp.zeros_like(out_ref)

    x = x_ref[...]
    g = jnp.dot(x, wi0_ref[0], preferred_element_type=jnp.float32)
    u = jnp.dot(x, wi1_ref[0], preferred_element_type=jnp.float32)
    h = jax.nn.silu(g) * u
    ye = jnp.dot(h, wo_ref[0], preferred_element_type=jnp.float32)

    lane = jax.lax.broadcasted_iota(jnp.int32, (T, E), 1)
    w_e = jnp.sum(jnp.where(lane == e, w_ref[...], 0.0), axis=-1, keepdims=True)
    out_ref[0] += ye * w_e


@functools.partial(jax.jit)
def kernel(x, router_logits, wi_0, wi_1, wo):
    parts = pl.pallas_call(
        _moe_body,
        grid=(NC, EC, NF),
        in_specs=[
            pl.BlockSpec((T, E), lambda c, e8, f: (0, 0)),
            pl.BlockSpec((T, H), lambda c, e8, f: (0, 0)),
            pl.BlockSpec((1, H, BF), lambda c, e8, f: (c * EC + e8, 0, f)),
            pl.BlockSpec((1, H, BF), lambda c, e8, f: (c * EC + e8, 0, f)),
            pl.BlockSpec((1, BF, H), lambda c, e8, f: (c * EC + e8, f, 0)),
        ],
        out_specs=pl.BlockSpec((1, T, H), lambda c, e8, f: (c, 0, 0)),
        out_shape=jax.ShapeDtypeStruct((NC, T, H), jnp.float32),
        scratch_shapes=[pltpu.VMEM((T, E), jnp.float32)],
        compiler_params=pltpu.CompilerParams(
            dimension_semantics=("parallel", "arbitrary", "arbitrary")),
    )(router_logits, x, wi_0, wi_1, wo)
    return parts[0] + parts[1]
